# baseline (speedup 1.0000x reference)
import functools

import jax
import jax.numpy as jnp
from jax import lax
from jax.experimental import pallas as pl
from jax.experimental.pallas import tpu as pltpu
from jax.experimental.pallas import tpu_sc as plsc

N = 10000
E = 320000
D = 128
DE = 16
NG = 64
NC, NS, L = 2, 16, 16
NW = NC * NS
K = 64
CHUNKS = 160
EPW = K * CHUNKS
E_PAD = EPW * NW
NROWS = 10112
RPT = NROWS // NS
DUMP = 10001
BE = 4096
BN = 1000
NBLK = N // BN


def _edge_body(h_hbm, c_hbm, src_hbm, dst_hbm, out_hbm,
               srcv, dstv, rows, cbuf, msg, sem_is, sem_id, sem_g, sem_c,
               sem_s, acc):
    cid = lax.axis_index("c")
    sid = lax.axis_index("s")
    wid = cid * NS + sid

    z16 = jnp.zeros((L,), jnp.float32)

    @plsc.parallel_loop(0, K)
    def _zero(i):
        for j in range(D // L):
            msg[i, pl.ds(j * L, L)] = z16

    r0 = sid * RPT
    for t in range(RPT // K):
        pltpu.sync_copy(msg, acc.at[pl.ds(r0 + t * K, K)])
    if RPT % K:
        pltpu.sync_copy(msg.at[pl.ds(0, RPT % K)],
                        acc.at[pl.ds(r0 + (RPT // K) * K, RPT % K)])
    plsc.subcore_barrier()

    ebase = wid * EPW

    def _issue_idx(g, b, which):
        base = pl.multiple_of(ebase + g * K, K)
        if which == 0:
            pltpu.async_copy(src_hbm.at[pl.ds(base, K)], srcv[b], sem_is[b])
        else:
            pltpu.async_copy(dst_hbm.at[pl.ds(base, K)], dstv[b], sem_id[b])

    def _wait_idx(g, b, which):
        base = pl.multiple_of(ebase + g * K, K)
        if which == 0:
            pltpu.make_async_copy(src_hbm.at[pl.ds(base, K)], srcv[b],
                                  sem_is[b]).wait()
        else:
            pltpu.make_async_copy(dst_hbm.at[pl.ds(base, K)], dstv[b],
                                  sem_id[b]).wait()

    def _issue_fetch(g, b):
        pltpu.async_copy(h_hbm.at[srcv[b]], rows[b], sem_g[b])
        base = pl.multiple_of(ebase + g * K, K)
        pltpu.async_copy(c_hbm.at[pl.ds(base, K)], cbuf[b], sem_c[b])

    def _wait_fetch(g, b):
        pltpu.make_async_copy(h_hbm.at[srcv[b]], rows[b], sem_g[b]).wait()
        base = pl.multiple_of(ebase + g * K, K)
        pltpu.make_async_copy(c_hbm.at[pl.ds(base, K)], cbuf[b],
                              sem_c[b]).wait()

    def _wait_scatter(b):
        pltpu.make_async_copy(msg, acc.at[dstv[b]], sem_s).wait()

    def _compute(b):
        @plsc.parallel_loop(0, K)
        def _inner(i):
            for j in range(D // L):
                s = pl.ds(j * L, L)
                msg[i, s] = jnp.maximum(rows[b][i, s] + cbuf[b][i, s], 0.0)

    def _step(g, b, pf_src, pf_dst):
        _wait_fetch(g, b)
        if pf_src:
            _issue_idx(g + 2, b, 0)
        _wait_scatter(1 - b)
        if pf_dst:
            _issue_idx(g + 1, 1 - b, 1)
        _compute(b)
        if pf_src:
            _wait_idx(g + 2, b, 0)
            _issue_fetch(g + 2, b)
        _wait_idx(g, b, 1)
        pltpu.async_copy(msg, acc.at[dstv[b]], sem_s, add=True)

    for b in range(2):
        base = pl.multiple_of(ebase + b * K, K)
        pltpu.sync_copy(src_hbm.at[pl.ds(base, K)], srcv[b])
        pltpu.sync_copy(dst_hbm.at[pl.ds(base, K)], dstv[b])
        pltpu.async_copy(h_hbm.at[srcv[b]], rows[b], sem_g[b])
        pltpu.async_copy(c_hbm.at[pl.ds(base, K)], cbuf[b], sem_c[b])

    _wait_fetch(0, 0)
    _issue_idx(2, 0, 0)
    _compute(0)
    _wait_idx(2, 0, 0)
    _issue_fetch(2, 0)
    pltpu.async_copy(msg, acc.at[dstv[0]], sem_s, add=True)

    _wait_fetch(1, 1)
    _issue_idx(3, 1, 0)
    _wait_scatter(0)
    _issue_idx(2, 0, 1)
    _compute(1)
    _wait_idx(3, 1, 0)
    _issue_fetch(3, 1)
    pltpu.async_copy(msg, acc.at[dstv[1]], sem_s, add=True)

    def _pair(i, carry):
        g = 2 * i
        _step(g, 0, pf_src=True, pf_dst=True)
        _step(g + 1, 1, pf_src=True, pf_dst=True)
        return carry

    lax.fori_loop(1, CHUNKS // 2 - 1, _pair, 0)

    _step(CHUNKS - 2, 0, pf_src=False, pf_dst=True)
    _step(CHUNKS - 1, 1, pf_src=False, pf_dst=False)
    _wait_scatter(1)
    plsc.subcore_barrier()

    bufs = (msg, rows[0], rows[1])
    for t in range(RPT // K):
        pltpu.sync_copy(acc.at[pl.ds(r0 + t * K, K)], bufs[t % 3])
        pltpu.sync_copy(bufs[t % 3], out_hbm.at[cid, pl.ds(r0 + t * K, K)])
    if RPT % K:
        tb = r0 + (RPT // K) * K
        pltpu.sync_copy(acc.at[pl.ds(tb, RPT % K)],
                        cbuf[0].at[pl.ds(0, RPT % K)])
        pltpu.sync_copy(cbuf[0].at[pl.ds(0, RPT % K)],
                        out_hbm.at[cid, pl.ds(tb, RPT % K)])


_edge_call = pl.kernel(
    _edge_body,
    out_type=jax.ShapeDtypeStruct((NC, NROWS, D), jnp.float32),
    mesh=plsc.VectorSubcoreMesh(core_axis_name="c", subcore_axis_name="s",
                                num_cores=NC, num_subcores=NS),
    scratch_types=[
        [pltpu.VMEM((K,), jnp.int32)] * 2,
        [pltpu.VMEM((K,), jnp.int32)] * 2,
        [pltpu.VMEM((K, D), jnp.float32)] * 2,
        [pltpu.VMEM((K, D), jnp.float32)] * 2,
        pltpu.VMEM((K, D), jnp.float32),
        [pltpu.SemaphoreType.DMA] * 2,
        [pltpu.SemaphoreType.DMA] * 2,
        [pltpu.SemaphoreType.DMA] * 2,
        [pltpu.SemaphoreType.DMA] * 2,
        pltpu.SemaphoreType.DMA,
        pltpu.VMEM_SHARED((NROWS, D), jnp.float32),
    ],
)


def _proj_body(ea_ref, w_ref, b_ref, c0_ref, c1_ref, c2_ref):
    c = jnp.dot(ea_ref[...], w_ref[...],
                preferred_element_type=jnp.float32) + b_ref[...]
    c0_ref[...] = c[:, 0:D]
    c1_ref[...] = c[:, D:2 * D]
    c2_ref[...] = c[:, 2 * D:3 * D]


@jax.jit
def _proj(ea, w, b):
    return pl.pallas_call(
        _proj_body,
        grid=(E_PAD // BE,),
        in_specs=[
            pl.BlockSpec((BE, DE), lambda i: (i, 0)),
            pl.BlockSpec((DE, 3 * D), lambda i: (0, 0)),
            pl.BlockSpec((1, 3 * D), lambda i: (0, 0)),
        ],
        out_specs=[pl.BlockSpec((BE, D), lambda i: (i, 0))] * 3,
        out_shape=[jax.ShapeDtypeStruct((E_PAD, D), jnp.float32)] * 3,
    )(ea, w, b)


def _mlp_body(h_ref, a_ref, pre_ref, sc_ref, w1_ref, b1_ref, w2_ref, b2_ref,
              z_ref):
    z0 = sc_ref[0, 0] * h_ref[...] + pre_ref[...] + a_ref[0] + a_ref[1]
    z1 = jnp.maximum(
        jnp.dot(z0, w1_ref[...], preferred_element_type=jnp.float32)
        + b1_ref[...], 0.0)
    z_ref[...] = jnp.dot(z1, w2_ref[...],
                         preferred_element_type=jnp.float32) + b2_ref[...]


@jax.jit
def _mlp(h, agg2, pre, scale, w1, b1, w2, b2):
    return pl.pallas_call(
        _mlp_body,
        grid=(NBLK,),
        in_specs=[
            pl.BlockSpec((BN, D), lambda i: (i, 0)),
            pl.BlockSpec((NC, BN, D), lambda i: (0, i, 0)),
            pl.BlockSpec((1, D), lambda i: (0, 0)),
            pl.BlockSpec(memory_space=pltpu.SMEM),
            pl.BlockSpec((D, D), lambda i: (0, 0)),
            pl.BlockSpec((1, D), lambda i: (0, 0)),
            pl.BlockSpec((D, D), lambda i: (0, 0)),
            pl.BlockSpec((1, D), lambda i: (0, 0)),
        ],
        out_specs=pl.BlockSpec((BN, D), lambda i: (i, 0)),
        out_shape=jax.ShapeDtypeStruct((N, D), jnp.float32),
    )(h, agg2, pre, scale, w1, b1, w2, b2)


def _ln(y, g, b):
    m = jnp.mean(y, axis=1, keepdims=True)
    d = y - m
    v = jnp.mean(d * d, axis=1, keepdims=True)
    return d * lax.rsqrt(v + 1e-5) * g + b


def _head_body(x0_ref, x1_ref, x2_ref, b_ref,
               wl1_ref, bl1_ref, lg1_ref, lb1_ref,
               wl2_ref, bl2_ref, lg2_ref, lb2_ref,
               ow_ref, ob_ref, out_ref, acc_ref):
    i = pl.program_id(0)

    @pl.when(i == 0)
    def _():
        acc_ref[...] = jnp.zeros((NG, 3 * D), jnp.float32)

    oh = (b_ref[...] == lax.broadcasted_iota(jnp.int32, (BN, NG), 1)
          ).astype(jnp.float32)
    dn = (((0,), (0,)), ((), ()))
    for t, xr in enumerate((x0_ref, x1_ref, x2_ref)):
        part = lax.dot_general(oh, xr[...], dimension_numbers=dn,
                               preferred_element_type=jnp.float32)
        sl = pl.ds(t * D, D)
        acc_ref[:, sl] = acc_ref[:, sl] + part

    @pl.when(i == NBLK - 1)
    def _():
        g = acc_ref[...]
        y = jnp.dot(g, wl1_ref[...], preferred_element_type=jnp.float32) \
            + bl1_ref[...]
        y = jnp.maximum(_ln(y, lg1_ref[...], lb1_ref[...]), 0.0)
        y = jnp.dot(y, wl2_ref[...], preferred_element_type=jnp.float32) \
            + bl2_ref[...]
        y = jnp.maximum(_ln(y, lg2_ref[...], lb2_ref[...]), 0.0)
        out_ref[...] = jnp.dot(y, ow_ref[...],
                               preferred_element_type=jnp.float32) \
            + ob_ref[...]


@jax.jit
def _head(x0, x1, x2, batch2d, wl1, bl1, lg1, lb1, wl2, bl2, lg2, lb2,
          ow, ob):
    full = lambda shape: pl.BlockSpec(shape, lambda i: tuple(0 for _ in shape))
    return pl.pallas_call(
        _head_body,
        grid=(NBLK,),
        in_specs=[
            pl.BlockSpec((BN, D), lambda i: (i, 0)),
            pl.BlockSpec((BN, D), lambda i: (i, 0)),
            pl.BlockSpec((BN, D), lambda i: (i, 0)),
            pl.BlockSpec((BN, 1), lambda i: (i, 0)),
            full((3 * D, 256)), full((1, 256)), full((1, 256)), full((1, 256)),
            full((256, D)), full((1, D)), full((1, D)), full((1, D)),
            full((D, 1)), full((1, 1)),
        ],
        out_specs=pl.BlockSpec((NG, 1), lambda i: (0, 0)),
        out_shape=jax.ShapeDtypeStruct((NG, 1), jnp.float32),
        scratch_shapes=[pltpu.VMEM((NG, 3 * D), jnp.float32)],
    )(x0, x1, x2, batch2d, wl1, bl1, lg1, lb1, wl2, bl2, lg2, lb2, ow, ob)


def kernel(x, edge_index, edge_attr, batch, params):
    pad = E_PAD - E
    srcp = jnp.concatenate([edge_index[0], jnp.zeros((pad,), jnp.int32)])
    dstp = jnp.concatenate([edge_index[1], jnp.full((pad,), DUMP, jnp.int32)])
    eap = jnp.concatenate([edge_attr, jnp.zeros((pad, DE), jnp.float32)])

    agg = params['agg']
    w_all = jnp.concatenate([p['edge'][0] for p in agg], axis=1)
    b_all = jnp.concatenate(
        [p['edge'][1] + params['vn_emb'][i][0] for i, p in enumerate(agg)]
    ).reshape(1, 3 * D)

    cs = _proj(eap, w_all, b_all)

    h = x
    xs = []
    for i, p in enumerate(agg):
        scale = (1.0 + p['eps']).reshape(1, 1)
        pre = (scale * params['vn_emb'][i]).astype(jnp.float32)
        agg2 = _edge_call(h, cs[i], srcp, dstp)
        (w1, b1), (w2, b2) = p['mlp']
        h = _mlp(h, agg2, pre, scale, w1, b1.reshape(1, D), w2,
                 b2.reshape(1, D))
        xs.append(h)

    (wl1, bl1), (lg1, lb1) = params['lin'][0]
    (wl2, bl2), (lg2, lb2) = params['lin'][1]
    ow, ob = params['out']
    return _head(xs[0], xs[1], xs[2], batch.reshape(N, 1),
                 wl1, bl1.reshape(1, 256), lg1.reshape(1, 256),
                 lb1.reshape(1, 256), wl2, bl2.reshape(1, D),
                 lg2.reshape(1, D), lb2.reshape(1, D), ow, ob.reshape(1, 1))


# speedup vs baseline: 2.3609x; 2.3609x over previous
import functools

import jax
import jax.numpy as jnp
from jax import lax
from jax.experimental import pallas as pl
from jax.experimental.pallas import tpu as pltpu
from jax.experimental.pallas import tpu_sc as plsc

N = 10000
E = 320000
D = 128
DE = 16
NG = 64
NC, NS, L = 2, 16, 16
NW = NC * NS
K = 64
CHUNKS = 160
CH0 = 48
CH1 = CHUNKS - CH0
EPW = K * CHUNKS
E_PAD = EPW * NS * 2 // 2 * 2
E_PAD = K * CHUNKS * NS * 2 // 2
E_PAD = 327680
NROWS = 10112
RPT = NROWS // NS
DUMP = 10001
BE = 4096
BN = 1000
NBLK = N // BN


def _edge_body(h_hbm, c_hbm, src_hbm, dst_hbm, out_hbm,
               srcv, dstv, rows, cbuf, msg, sem_is, sem_id, sem_g, sem_c,
               sem_s, acc):
    cid = lax.axis_index("c")
    sid = lax.axis_index("s")

    z16 = jnp.zeros((L,), jnp.float32)

    @plsc.parallel_loop(0, K)
    def _zero(i):
        for j in range(D // L):
            msg[i, pl.ds(j * L, L)] = z16

    r0 = sid * RPT
    for t in range(RPT // K):
        pltpu.sync_copy(msg, acc.at[pl.ds(r0 + t * K, K)])
    if RPT % K:
        pltpu.sync_copy(msg.at[pl.ds(0, RPT % K)],
                        acc.at[pl.ds(r0 + (RPT // K) * K, RPT % K)])
    plsc.subcore_barrier()

    nch = jnp.where(cid == 0, CH0, CH1)
    ebase = jnp.where(cid == 0, sid * (CH0 * K),
                      NS * CH0 * K + sid * (CH1 * K))

    def _issue_idx(g, b, which):
        base = pl.multiple_of(ebase + g * K, K)
        if which == 0:
            pltpu.async_copy(src_hbm.at[pl.ds(base, K)], srcv[b], sem_is[b])
        else:
            pltpu.async_copy(dst_hbm.at[pl.ds(base, K)], dstv[b], sem_id[b])

    def _wait_idx(g, b, which):
        base = pl.multiple_of(ebase + g * K, K)
        if which == 0:
            pltpu.make_async_copy(src_hbm.at[pl.ds(base, K)], srcv[b],
                                  sem_is[b]).wait()
        else:
            pltpu.make_async_copy(dst_hbm.at[pl.ds(base, K)], dstv[b],
                                  sem_id[b]).wait()

    def _issue_fetch(g, b):
        pltpu.async_copy(h_hbm.at[srcv[b]], rows[b], sem_g[b])
        base = pl.multiple_of(ebase + g * K, K)
        pltpu.async_copy(c_hbm.at[pl.ds(base, K)], cbuf[b], sem_c[b])

    def _wait_fetch(g, b):
        pltpu.make_async_copy(h_hbm.at[srcv[b]], rows[b], sem_g[b]).wait()
        base = pl.multiple_of(ebase + g * K, K)
        pltpu.make_async_copy(c_hbm.at[pl.ds(base, K)], cbuf[b],
                              sem_c[b]).wait()

    def _wait_scatter(b):
        pltpu.make_async_copy(msg, acc.at[dstv[b]], sem_s).wait()

    def _compute(b):
        @plsc.parallel_loop(0, K)
        def _inner(i):
            for j in range(D // L):
                s = pl.ds(j * L, L)
                msg[i, s] = jnp.maximum(rows[b][i, s] + cbuf[b][i, s], 0.0)

    def _step(g, b, pf_src, pf_dst):
        _wait_fetch(g, b)
        if pf_src:
            _issue_idx(g + 2, b, 0)
        _wait_scatter(1 - b)
        if pf_dst:
            _issue_idx(g + 1, 1 - b, 1)
        _compute(b)
        if pf_src:
            _wait_idx(g + 2, b, 0)
            _issue_fetch(g + 2, b)
        _wait_idx(g, b, 1)
        pltpu.async_copy(msg, acc.at[dstv[b]], sem_s, add=True)

    for b in range(2):
        base = pl.multiple_of(ebase + b * K, K)
        pltpu.sync_copy(src_hbm.at[pl.ds(base, K)], srcv[b])
        pltpu.sync_copy(dst_hbm.at[pl.ds(base, K)], dstv[b])
        pltpu.async_copy(h_hbm.at[srcv[b]], rows[b], sem_g[b])
        pltpu.async_copy(c_hbm.at[pl.ds(base, K)], cbuf[b], sem_c[b])

    _wait_fetch(0, 0)
    _issue_idx(2, 0, 0)
    _compute(0)
    _wait_idx(2, 0, 0)
    _issue_fetch(2, 0)
    pltpu.async_copy(msg, acc.at[dstv[0]], sem_s, add=True)

    _wait_fetch(1, 1)
    _issue_idx(3, 1, 0)
    _wait_scatter(0)
    _issue_idx(2, 0, 1)
    _compute(1)
    _wait_idx(3, 1, 0)
    _issue_fetch(3, 1)
    pltpu.async_copy(msg, acc.at[dstv[1]], sem_s, add=True)

    def _pair(i, carry):
        g = 2 * i
        _step(g, 0, pf_src=True, pf_dst=True)
        _step(g + 1, 1, pf_src=True, pf_dst=True)
        return carry

    lax.fori_loop(1, nch // 2 - 1, _pair, 0)

    _step(nch - 2, 0, pf_src=False, pf_dst=True)
    _step(nch - 1, 1, pf_src=False, pf_dst=False)
    _wait_scatter(1)
    plsc.subcore_barrier()

    bufs = (msg, rows[0], rows[1])
    for t in range(RPT // K):
        pltpu.sync_copy(acc.at[pl.ds(r0 + t * K, K)], bufs[t % 3])
        pltpu.sync_copy(bufs[t % 3], out_hbm.at[cid, pl.ds(r0 + t * K, K)])
    if RPT % K:
        tb = r0 + (RPT // K) * K
        pltpu.sync_copy(acc.at[pl.ds(tb, RPT % K)],
                        cbuf[0].at[pl.ds(0, RPT % K)])
        pltpu.sync_copy(cbuf[0].at[pl.ds(0, RPT % K)],
                        out_hbm.at[cid, pl.ds(tb, RPT % K)])


_edge_call = pl.kernel(
    _edge_body,
    out_type=jax.ShapeDtypeStruct((NC, NROWS, D), jnp.float32),
    mesh=plsc.VectorSubcoreMesh(core_axis_name="c", subcore_axis_name="s",
                                num_cores=NC, num_subcores=NS),
    scratch_types=[
        [pltpu.VMEM((K,), jnp.int32)] * 2,
        [pltpu.VMEM((K,), jnp.int32)] * 2,
        [pltpu.VMEM((K, D), jnp.float32)] * 2,
        [pltpu.VMEM((K, D), jnp.float32)] * 2,
        pltpu.VMEM((K, D), jnp.float32),
        [pltpu.SemaphoreType.DMA] * 2,
        [pltpu.SemaphoreType.DMA] * 2,
        [pltpu.SemaphoreType.DMA] * 2,
        [pltpu.SemaphoreType.DMA] * 2,
        pltpu.SemaphoreType.DMA,
        pltpu.VMEM_SHARED((NROWS, D), jnp.float32),
    ],
)


def _proj_body(ea_ref, w_ref, b_ref, c0_ref, c1_ref, c2_ref):
    c = jnp.dot(ea_ref[...], w_ref[...],
                preferred_element_type=jnp.float32) + b_ref[...]
    c0_ref[...] = c[:, 0:D]
    c1_ref[...] = c[:, D:2 * D]
    c2_ref[...] = c[:, 2 * D:3 * D]


@jax.jit
def _proj(ea, w, b):
    return pl.pallas_call(
        _proj_body,
        grid=(E_PAD // BE,),
        in_specs=[
            pl.BlockSpec((BE, DE), lambda i: (i, 0)),
            pl.BlockSpec((DE, 3 * D), lambda i: (0, 0)),
            pl.BlockSpec((1, 3 * D), lambda i: (0, 0)),
        ],
        out_specs=[pl.BlockSpec((BE, D), lambda i: (i, 0))] * 3,
        out_shape=[jax.ShapeDtypeStruct((E_PAD, D), jnp.float32)] * 3,
    )(ea, w, b)


def _mlp_body(h_ref, a_ref, pre_ref, sc_ref, w1_ref, b1_ref, w2_ref, b2_ref,
              z_ref):
    z0 = sc_ref[0, 0] * h_ref[...] + pre_ref[...] + a_ref[0] + a_ref[1]
    z1 = jnp.maximum(
        jnp.dot(z0, w1_ref[...], preferred_element_type=jnp.float32)
        + b1_ref[...], 0.0)
    z_ref[...] = jnp.dot(z1, w2_ref[...],
                         preferred_element_type=jnp.float32) + b2_ref[...]


@jax.jit
def _mlp(h, agg2, pre, scale, w1, b1, w2, b2):
    return pl.pallas_call(
        _mlp_body,
        grid=(NBLK,),
        in_specs=[
            pl.BlockSpec((BN, D), lambda i: (i, 0)),
            pl.BlockSpec((NC, BN, D), lambda i: (0, i, 0)),
            pl.BlockSpec((1, D), lambda i: (0, 0)),
            pl.BlockSpec(memory_space=pltpu.SMEM),
            pl.BlockSpec((D, D), lambda i: (0, 0)),
            pl.BlockSpec((1, D), lambda i: (0, 0)),
            pl.BlockSpec((D, D), lambda i: (0, 0)),
            pl.BlockSpec((1, D), lambda i: (0, 0)),
        ],
        out_specs=pl.BlockSpec((BN, D), lambda i: (i, 0)),
        out_shape=jax.ShapeDtypeStruct((N, D), jnp.float32),
    )(h, agg2, pre, scale, w1, b1, w2, b2)


def _ln(y, g, b):
    m = jnp.mean(y, axis=1, keepdims=True)
    d = y - m
    v = jnp.mean(d * d, axis=1, keepdims=True)
    return d * lax.rsqrt(v + 1e-5) * g + b


def _head_body(x0_ref, x1_ref, x2_ref, b_ref,
               wl1_ref, bl1_ref, lg1_ref, lb1_ref,
               wl2_ref, bl2_ref, lg2_ref, lb2_ref,
               ow_ref, ob_ref, out_ref, acc_ref):
    i = pl.program_id(0)

    @pl.when(i == 0)
    def _():
        acc_ref[...] = jnp.zeros((NG, 3 * D), jnp.float32)

    oh = (b_ref[...] == lax.broadcasted_iota(jnp.int32, (BN, NG), 1)
          ).astype(jnp.float32)
    dn = (((0,), (0,)), ((), ()))
    for t, xr in enumerate((x0_ref, x1_ref, x2_ref)):
        part = lax.dot_general(oh, xr[...], dimension_numbers=dn,
                               preferred_element_type=jnp.float32)
        sl = pl.ds(t * D, D)
        acc_ref[:, sl] = acc_ref[:, sl] + part

    @pl.when(i == NBLK - 1)
    def _():
        g = acc_ref[...]
        y = jnp.dot(g, wl1_ref[...], preferred_element_type=jnp.float32) \
            + bl1_ref[...]
        y = jnp.maximum(_ln(y, lg1_ref[...], lb1_ref[...]), 0.0)
        y = jnp.dot(y, wl2_ref[...], preferred_element_type=jnp.float32) \
            + bl2_ref[...]
        y = jnp.maximum(_ln(y, lg2_ref[...], lb2_ref[...]), 0.0)
        out_ref[...] = jnp.dot(y, ow_ref[...],
                               preferred_element_type=jnp.float32) \
            + ob_ref[...]


@jax.jit
def _head(x0, x1, x2, batch2d, wl1, bl1, lg1, lb1, wl2, bl2, lg2, lb2,
          ow, ob):
    full = lambda shape: pl.BlockSpec(shape, lambda i: tuple(0 for _ in shape))
    return pl.pallas_call(
        _head_body,
        grid=(NBLK,),
        in_specs=[
            pl.BlockSpec((BN, D), lambda i: (i, 0)),
            pl.BlockSpec((BN, D), lambda i: (i, 0)),
            pl.BlockSpec((BN, D), lambda i: (i, 0)),
            pl.BlockSpec((BN, 1), lambda i: (i, 0)),
            full((3 * D, 256)), full((1, 256)), full((1, 256)), full((1, 256)),
            full((256, D)), full((1, D)), full((1, D)), full((1, D)),
            full((D, 1)), full((1, 1)),
        ],
        out_specs=pl.BlockSpec((NG, 1), lambda i: (0, 0)),
        out_shape=jax.ShapeDtypeStruct((NG, 1), jnp.float32),
        scratch_shapes=[pltpu.VMEM((NG, 3 * D), jnp.float32)],
    )(x0, x1, x2, batch2d, wl1, bl1, lg1, lb1, wl2, bl2, lg2, lb2, ow, ob)


def kernel(x, edge_index, edge_attr, batch, params):
    pad = E_PAD - E
    srcp = jnp.concatenate([edge_index[0], jnp.zeros((pad,), jnp.int32)])
    dstp = jnp.concatenate([edge_index[1], jnp.full((pad,), DUMP, jnp.int32)])
    eap = jnp.concatenate([edge_attr, jnp.zeros((pad, DE), jnp.float32)])

    agg = params['agg']
    w_all = jnp.concatenate([p['edge'][0] for p in agg], axis=1)
    b_all = jnp.concatenate(
        [p['edge'][1] + params['vn_emb'][i][0] for i, p in enumerate(agg)]
    ).reshape(1, 3 * D)

    cs = _proj(eap, w_all, b_all)

    h = x
    xs = []
    for i, p in enumerate(agg):
        scale = (1.0 + p['eps']).reshape(1, 1)
        pre = (scale * params['vn_emb'][i]).astype(jnp.float32)
        agg2 = _edge_call(h, cs[i], srcp, dstp)
        (w1, b1), (w2, b2) = p['mlp']
        h = _mlp(h, agg2, pre, scale, w1, b1.reshape(1, D), w2,
                 b2.reshape(1, D))
        xs.append(h)

    (wl1, bl1), (lg1, lb1) = params['lin'][0]
    (wl2, bl2), (lg2, lb2) = params['lin'][1]
    ow, ob = params['out']
    return _head(xs[0], xs[1], xs[2], batch.reshape(N, 1),
                 wl1, bl1.reshape(1, 256), lg1.reshape(1, 256),
                 lb1.reshape(1, 256), wl2, bl2.reshape(1, D),
                 lg2.reshape(1, D), lb2.reshape(1, D), ow, ob.reshape(1, 1))
